# baseline (speedup 1.0000x reference)
import functools

import jax
import jax.numpy as jnp
from jax import lax
from jax.experimental import pallas as pl
from jax.experimental.pallas import tpu as pltpu
from jax.experimental.pallas import tpu_sc as plsc

N = 16384
D = 128
V = 1000
TROWS = 64
CHUNK = 128
NC = 2
NS = 16
NW = NC * NS
ROWS_PER_W = N // NW
CHUNKS_PER_W = ROWS_PER_W // CHUNK
NIDX_ROWS = N // CHUNK

_mesh = plsc.VectorSubcoreMesh(core_axis_name="c", subcore_axis_name="s")


@functools.partial(
    pl.kernel,
    mesh=_mesh,
    out_type=jax.ShapeDtypeStruct((N, 2 * D), jnp.float32),
    scratch_types=[
        pltpu.VMEM((CHUNKS_PER_W, CHUNK), jnp.int32),
        pltpu.VMEM((ROWS_PER_W, D), jnp.float32),
        pltpu.VMEM((2, CHUNK, D), jnp.float32),
        pltpu.VMEM_SHARED((V, D), jnp.float32),
        pltpu.SemaphoreType.DMA,
        pltpu.SemaphoreType.DMA,
        pltpu.SemaphoreType.DMA,
    ],
)
def _emb_concat(
    x_hbm, y_hbm, emb_hbm, out_hbm, idx_v, xbuf, gbuf, tab_sh, gsem, xsem, wsem
):
    sid = lax.axis_index("s")
    wid = sid * NC + lax.axis_index("c")
    base = wid * ROWS_PER_W
    trow = jnp.minimum(sid * TROWS, V - TROWS)
    pltpu.sync_copy(
        emb_hbm.at[pl.ds(trow, TROWS)],
        tab_sh.at[pl.ds(trow, TROWS)],
    )
    pltpu.sync_copy(y_hbm.at[pl.ds(wid * CHUNKS_PER_W, CHUNKS_PER_W)], idx_v)
    plsc.subcore_barrier()

    cx = pltpu.async_copy(x_hbm.at[pl.ds(base, ROWS_PER_W)], xbuf, xsem)

    NBUF = 2
    loads_g = [None] * CHUNKS_PER_W
    writes = [None] * CHUNKS_PER_W

    def fire_write(j):
        loads_g[j].wait()
        writes[j] = pltpu.async_copy(
            gbuf.at[j % NBUF],
            out_hbm.at[pl.ds(base + j * CHUNK, CHUNK), pl.ds(D, D)],
            wsem,
        )

    for j in range(CHUNKS_PER_W):
        if j >= NBUF:
            writes[j - NBUF].wait()
        loads_g[j] = pltpu.async_copy(tab_sh.at[idx_v.at[j]], gbuf.at[j % NBUF], gsem)
        if j >= 1:
            fire_write(j - 1)
    fire_write(CHUNKS_PER_W - 1)
    cx.wait()
    cxw = pltpu.async_copy(
        xbuf, out_hbm.at[pl.ds(base, ROWS_PER_W), pl.ds(0, D)], xsem
    )
    for j in range(max(0, CHUNKS_PER_W - NBUF), CHUNKS_PER_W):
        writes[j].wait()
    cxw.wait()


def kernel(x, y, embedding):
    y2d = y.astype(jnp.int32).reshape(NIDX_ROWS, CHUNK)
    return _emb_concat(x, y2d, embedding)


# speedup vs baseline: 1.0166x; 1.0166x over previous
import functools

import jax
import jax.numpy as jnp
from jax import lax
from jax.experimental import pallas as pl
from jax.experimental.pallas import tpu as pltpu
from jax.experimental.pallas import tpu_sc as plsc

N = 16384
D = 128
V = 1000
TROWS = 64
CHUNK = 128
NC = 2
NS = 16
NW = NC * NS
ROWS_PER_W = N // NW
CHUNKS_PER_W = ROWS_PER_W // CHUNK
NIDX_ROWS = N // CHUNK

_mesh = plsc.VectorSubcoreMesh(core_axis_name="c", subcore_axis_name="s")


@functools.partial(
    pl.kernel,
    mesh=_mesh,
    out_type=jax.ShapeDtypeStruct((N, 2 * D), jnp.float32),
    scratch_types=[
        pltpu.VMEM((CHUNKS_PER_W, CHUNK), jnp.int32),
        pltpu.VMEM((3, 2, CHUNK, D), jnp.float32),
        pltpu.VMEM_SHARED((V, D), jnp.float32),
        pltpu.SemaphoreType.DMA,
        pltpu.SemaphoreType.DMA,
        pltpu.SemaphoreType.DMA,
    ],
)
def _emb_concat(x_hbm, y_hbm, emb_hbm, out_hbm, idx_v, obuf, tab_sh, gsem, xsem, wsem):
    sid = lax.axis_index("s")
    wid = sid * NC + lax.axis_index("c")
    base = wid * ROWS_PER_W
    trow = jnp.minimum(sid * TROWS, V - TROWS)
    pltpu.sync_copy(
        emb_hbm.at[pl.ds(trow, TROWS)],
        tab_sh.at[pl.ds(trow, TROWS)],
    )
    pltpu.sync_copy(y_hbm.at[pl.ds(wid * CHUNKS_PER_W, CHUNKS_PER_W)], idx_v)
    plsc.subcore_barrier()

    NBUF = 3
    loads_g = [None] * CHUNKS_PER_W
    loads_x = [None] * CHUNKS_PER_W
    writes = [None] * CHUNKS_PER_W

    def fire_writes(j):
        b = j % NBUF
        loads_g[j].wait()
        loads_x[j].wait()
        r0 = pl.ds(base + j * CHUNK, CHUNK)
        writes[j] = (
            pltpu.async_copy(obuf.at[b, 0], out_hbm.at[r0, pl.ds(0, D)], wsem),
            pltpu.async_copy(obuf.at[b, 1], out_hbm.at[r0, pl.ds(D, D)], wsem),
        )

    for j in range(CHUNKS_PER_W):
        b = j % NBUF
        if j >= NBUF:
            for c in writes[j - NBUF]:
                c.wait()
        loads_g[j] = pltpu.async_copy(tab_sh.at[idx_v.at[j]], obuf.at[b, 1], gsem)
        loads_x[j] = pltpu.async_copy(
            x_hbm.at[pl.ds(base + j * CHUNK, CHUNK)], obuf.at[b, 0], xsem
        )
        if j >= 1:
            fire_writes(j - 1)
    fire_writes(CHUNKS_PER_W - 1)
    for j in range(max(0, CHUNKS_PER_W - NBUF), CHUNKS_PER_W):
        for c in writes[j]:
            c.wait()


def kernel(x, y, embedding):
    y2d = y.astype(jnp.int32).reshape(NIDX_ROWS, CHUNK)
    return _emb_concat(x, y2d, embedding)


# speedup vs baseline: 1.0399x; 1.0229x over previous
import functools

import jax
import jax.numpy as jnp
from jax import lax
from jax.experimental import pallas as pl
from jax.experimental.pallas import tpu as pltpu
from jax.experimental.pallas import tpu_sc as plsc

N = 16384
D = 128
V = 1000
TROWS = 64
CHUNK = 128
NC = 2
NS = 16
NW = NC * NS
ROWS_PER_W = N // NW
CHUNKS_PER_W = ROWS_PER_W // CHUNK
NIDX_ROWS = N // CHUNK

_mesh = plsc.VectorSubcoreMesh(core_axis_name="c", subcore_axis_name="s")


@functools.partial(
    pl.kernel,
    mesh=_mesh,
    out_type=jax.ShapeDtypeStruct((N, 2 * D), jnp.float32),
    scratch_types=[
        pltpu.VMEM((CHUNKS_PER_W, CHUNK), jnp.int32),
        pltpu.VMEM((3, 2, CHUNK, D), jnp.float32),
        pltpu.VMEM_SHARED((V, D), jnp.float32),
        pltpu.SemaphoreType.DMA,
        pltpu.SemaphoreType.DMA,
        pltpu.SemaphoreType.DMA,
    ],
)
def _emb_concat(x_hbm, y_hbm, emb_hbm, out_hbm, idx_v, obuf, tab_sh, gsem, xsem, wsem):
    sid = lax.axis_index("s")
    wid = sid * NC + lax.axis_index("c")
    base = wid * ROWS_PER_W
    trow = jnp.minimum(sid * TROWS, V - TROWS)
    ct = pltpu.async_copy(
        emb_hbm.at[pl.ds(trow, TROWS)], tab_sh.at[pl.ds(trow, TROWS)], xsem
    )
    ci = pltpu.async_copy(
        y_hbm.at[pl.ds(wid * CHUNKS_PER_W, CHUNKS_PER_W)], idx_v, gsem
    )
    ct.wait()
    ci.wait()
    plsc.subcore_barrier()

    NBUF = 3
    loads_g = [None] * CHUNKS_PER_W
    loads_x = [None] * CHUNKS_PER_W
    writes = [None] * CHUNKS_PER_W

    def fire_writes(j):
        b = j % NBUF
        loads_g[j].wait()
        loads_x[j].wait()
        r0 = pl.ds(base + j * CHUNK, CHUNK)
        writes[j] = (
            pltpu.async_copy(obuf.at[b, 0], out_hbm.at[r0, pl.ds(0, D)], wsem),
            pltpu.async_copy(obuf.at[b, 1], out_hbm.at[r0, pl.ds(D, D)], wsem),
        )

    for j in range(CHUNKS_PER_W):
        b = j % NBUF
        if j >= NBUF:
            for c in writes[j - NBUF]:
                c.wait()
        loads_x[j] = pltpu.async_copy(
            x_hbm.at[pl.ds(base + j * CHUNK, CHUNK)], obuf.at[b, 0], xsem
        )
        loads_g[j] = pltpu.async_copy(tab_sh.at[idx_v.at[j]], obuf.at[b, 1], gsem)
        if j >= 1:
            fire_writes(j - 1)
    fire_writes(CHUNKS_PER_W - 1)
    for j in range(max(0, CHUNKS_PER_W - NBUF), CHUNKS_PER_W):
        for c in writes[j]:
            c.wait()


def kernel(x, y, embedding):
    y2d = y.astype(jnp.int32).reshape(NIDX_ROWS, CHUNK)
    return _emb_concat(x, y2d, embedding)


# speedup vs baseline: 1.0795x; 1.0380x over previous
import functools

import jax
import jax.numpy as jnp
from jax import lax
from jax.experimental import pallas as pl
from jax.experimental.pallas import tpu as pltpu
from jax.experimental.pallas import tpu_sc as plsc

N = 16384
D = 128
V = 1000
TROWS = 64
CHUNK = 128
NC = 2
NS = 16
NW = NC * NS
ROWS_PER_W = N // NW
CHUNKS_PER_W = ROWS_PER_W // CHUNK
NIDX_ROWS = N // CHUNK

_mesh = plsc.VectorSubcoreMesh(core_axis_name="c", subcore_axis_name="s")


@functools.partial(
    pl.kernel,
    mesh=_mesh,
    out_type=jax.ShapeDtypeStruct((N, 2 * D), jnp.float32),
    scratch_types=[
        pltpu.VMEM((CHUNKS_PER_W, CHUNK), jnp.int32),
        pltpu.VMEM((3, 2, CHUNK, D), jnp.float32),
        pltpu.VMEM_SHARED((V, D), jnp.float32),
        pltpu.SemaphoreType.DMA,
        pltpu.SemaphoreType.DMA,
        pltpu.SemaphoreType.DMA,
    ],
)
def _emb_concat(x_hbm, y_hbm, emb_hbm, out_hbm, idx_v, obuf, tab_sh, gsem, xsem, wsem):
    sid = lax.axis_index("s")
    wid = sid * NC + lax.axis_index("c")
    base = wid * ROWS_PER_W
    trow = jnp.minimum(sid * TROWS, V - TROWS)
    ct = pltpu.async_copy(
        emb_hbm.at[pl.ds(trow, TROWS)], tab_sh.at[pl.ds(trow, TROWS)], xsem
    )
    ci = pltpu.async_copy(
        y_hbm.at[pl.ds(wid * CHUNKS_PER_W, CHUNKS_PER_W)], idx_v, gsem
    )
    NBUF = 3
    loads_g = [None] * CHUNKS_PER_W
    loads_x = [None] * CHUNKS_PER_W
    writes = [None] * CHUNKS_PER_W

    for j in range(min(NBUF, CHUNKS_PER_W)):
        loads_x[j] = pltpu.async_copy(
            x_hbm.at[pl.ds(base + j * CHUNK, CHUNK)], obuf.at[j % NBUF, 0], xsem
        )
    ct.wait()
    plsc.subcore_barrier()
    ci.wait()

    def fire_writes(j):
        b = j % NBUF
        loads_g[j].wait()
        loads_x[j].wait()
        r0 = pl.ds(base + j * CHUNK, CHUNK)
        writes[j] = (
            pltpu.async_copy(obuf.at[b, 0], out_hbm.at[r0, pl.ds(0, D)], wsem),
            pltpu.async_copy(obuf.at[b, 1], out_hbm.at[r0, pl.ds(D, D)], wsem),
        )

    for j in range(CHUNKS_PER_W):
        b = j % NBUF
        if j >= NBUF:
            for c in writes[j - NBUF]:
                c.wait()
            loads_x[j] = pltpu.async_copy(
                x_hbm.at[pl.ds(base + j * CHUNK, CHUNK)], obuf.at[b, 0], xsem
            )
        loads_g[j] = pltpu.async_copy(tab_sh.at[idx_v.at[j]], obuf.at[b, 1], gsem)
        if j >= 1:
            fire_writes(j - 1)
    fire_writes(CHUNKS_PER_W - 1)
    for j in range(max(0, CHUNKS_PER_W - NBUF), CHUNKS_PER_W):
        for c in writes[j]:
            c.wait()


def kernel(x, y, embedding):
    y2d = y.astype(jnp.int32).reshape(NIDX_ROWS, CHUNK)
    return _emb_concat(x, y2d, embedding)
